# baseline (speedup 1.0000x reference)
import functools

import jax
import jax.numpy as jnp
from jax import lax
from jax.experimental import pallas as pl
from jax.experimental.pallas import tpu as pltpu
from jax.experimental.pallas import tpu_sc as plsc

B, C, H, W = 64, 384, 32, 32
HW = H * W
E = 8
RED = 48
L = 16

NC = 2
NS = 16
NW = NC * NS
BPW = B // NW
CS = 32
NCH = C // CS
UNROLL = 4


def _sc_body(x_hbm, w1_hbm, b1_hbm, w2p_hbm, b2p_hbm, out_hbm,
             xb0, xb1, pbuf, w1v, b1v, w2v, b2v, obuf,
             csem0, csem1):
    wid = lax.axis_index("s") * NC + lax.axis_index("c")

    pltpu.sync_copy(w1_hbm, w1v)
    pltpu.sync_copy(b1_hbm, b1v)
    pltpu.sync_copy(w2p_hbm, w2v)
    pltpu.sync_copy(b2p_hbm, b2v)

    cvecs = [lax.iota(jnp.int32, L) + g * L for g in range(CS // L)]
    zero = jnp.zeros((L,), jnp.float32)

    def start(b, ci, buf, sem):
        pltpu.make_async_copy(
            x_hbm.at[b, pl.ds(ci * CS, CS)], buf, sem).start()

    def accumulate(buf, c0):
        for g, cvec in enumerate(cvecs):
            def pos(tq, carry, cvec=cvec):
                accs = list(carry)
                for p in range(UNROLL):
                    t = tq * UNROLL + p
                    hh = jnp.full((L,), t >> 5, jnp.int32)
                    ww = jnp.full((L,), t & (W - 1), jnp.int32)
                    accs[p] = accs[p] + plsc.load_gather(
                        buf, [cvec, hh, ww])
                return tuple(accs)
            a = lax.fori_loop(0, HW // UNROLL, pos, (zero,) * UNROLL,
                              unroll=False)
            pbuf[pl.ds(c0 + g * L, L)] = (a[0] + a[1]) + (a[2] + a[3])

    for bl in range(BPW):
        b = wid * BPW + bl
        if bl == 0:
            start(b, 0, xb0, csem0)

        def chunk_pair(g2, _, b=b):
            ci = 2 * g2

            @pl.when(ci + 1 < NCH)
            def _():
                start(b, ci + 1, xb1, csem1)

            pltpu.make_async_copy(
                x_hbm.at[b, pl.ds(ci * CS, CS)], xb0, csem0).wait()
            accumulate(xb0, ci * CS)

            @pl.when(ci + 2 < NCH)
            def _():
                start(b, ci + 2, xb0, csem0)

            pltpu.make_async_copy(
                x_hbm.at[b, pl.ds((ci + 1) * CS, CS)], xb1, csem1).wait()
            accumulate(xb1, (ci + 1) * CS)
            return 0

        lax.fori_loop(0, NCH // 2, chunk_pair, 0, unroll=False)

        if bl + 1 < BPW:
            start(b + 1, 0, xb0, csem0)

        inv = 1.0 / HW

        def h_acc(q, carry):
            hvs = list(carry)
            pv = pbuf[pl.ds(q * L, L)]
            for k in range(L):
                s = pv[k] * inv
                for j in range(RED // L):
                    hvs[j] = hvs[j] + s * w1v[q * L + k, pl.ds(j * L, L)]
            return tuple(hvs)

        hvs = lax.fori_loop(0, C // L, h_acc, (zero,) * (RED // L),
                            unroll=False)
        hjs = []
        for j in range(RED // L):
            hj = hvs[j] + b1v[pl.ds(j * L, L)]
            hj = hj / (1.0 + jnp.exp(-hj))
            hjs.append(hj)

        logits = b2v[...]
        for j in range(RED // L):
            for k in range(L):
                logits = logits + hjs[j][k] * w2v[j * L + k]

        ex = jnp.exp(logits)
        es = [ex[k] for k in range(E)]
        tot = es[0]
        for k in range(1, E):
            tot = tot + es[k]
        rtot = (jnp.ones((L,), jnp.float32) / jnp.full((L,), tot))[0]
        ws = [e * rtot for e in es]
        m1 = ws[0]
        for k in range(1, E):
            m1 = jnp.maximum(m1, ws[k])
        i1 = jnp.int32(E - 1)
        for k in reversed(range(E)):
            i1 = jnp.where(ws[k] == m1, jnp.int32(k), i1)
        wr = [jnp.where(i1 == k, -1.0, ws[k]) for k in range(E)]
        m2 = wr[0]
        for k in range(1, E):
            m2 = jnp.maximum(m2, wr[k])
        i2 = jnp.int32(E - 1)
        for k in reversed(range(E)):
            i2 = jnp.where(wr[k] == m2, jnp.int32(k), i2)
        keep = [(i1 == k) | (i2 == k) for k in range(E)]
        wm = [jnp.where(keep[k], ws[k], 0.0) for k in range(E)]
        den = wm[0]
        for k in range(1, E):
            den = den + wm[k]
        rden = (jnp.ones((L,), jnp.float32)
                / jnp.full((L,), den + 1e-8))[0]

        for e in range(E):
            splat = jnp.full((L,), wm[e] * rden, jnp.float32)

            def fill(h, _, e=e, splat=splat):
                obuf[e, h, pl.ds(0, L)] = splat
                obuf[e, h, pl.ds(L, L)] = splat
                return 0
            lax.fori_loop(0, H, fill, 0, unroll=False)
        pltpu.sync_copy(obuf, out_hbm.at[b])


@jax.jit
def kernel(x, W1, b1, W2, b2):
    w2p = jnp.pad(W2, ((0, 0), (0, L - E)))
    b2p = jnp.concatenate([b2, jnp.full((L - E,), -1e30, jnp.float32)])
    mesh = plsc.VectorSubcoreMesh(core_axis_name="c", subcore_axis_name="s")
    sck = functools.partial(
        pl.kernel,
        out_type=jax.ShapeDtypeStruct((B, E, H, W), jnp.float32),
        mesh=mesh,
        compiler_params=pltpu.CompilerParams(
            needs_layout_passes=False, use_tc_tiling_on_sc=False),
        scratch_types=[
            pltpu.VMEM((CS, H, W), jnp.float32),
            pltpu.VMEM((CS, H, W), jnp.float32),
            pltpu.VMEM((C,), jnp.float32),
            pltpu.VMEM((C, RED), jnp.float32),
            pltpu.VMEM((RED,), jnp.float32),
            pltpu.VMEM((RED, L), jnp.float32),
            pltpu.VMEM((L,), jnp.float32),
            pltpu.VMEM((E, H, W), jnp.float32),
            pltpu.SemaphoreType.DMA,
            pltpu.SemaphoreType.DMA,
        ],
    )(_sc_body)
    return sck(x, W1, b1, w2p, b2p)


# speedup vs baseline: 7.3647x; 7.3647x over previous
import jax
import jax.numpy as jnp
from jax import lax
from jax.experimental import pallas as pl

B, C, H, W = 64, 384, 32, 32
HW = H * W
E = 8
RED = 48


def _body(x_ref, w1_ref, b1_ref, w2_ref, b2_ref, out_ref):
    xs = x_ref[0]
    pooled = jnp.sum(xs, axis=(0, 1)).reshape(1, C) * (1.0 / HW)
    h = jnp.dot(pooled, w1_ref[...], preferred_element_type=jnp.float32)
    h = h + b1_ref[...]
    h = h * jax.nn.sigmoid(h)
    logits = jnp.dot(h, w2_ref[...], preferred_element_type=jnp.float32)
    logits = logits + b2_ref[...]
    w = jax.nn.softmax(logits, axis=1)
    idx = lax.broadcasted_iota(jnp.int32, (1, E), 1)
    m1 = jnp.max(w, axis=1, keepdims=True)
    i1 = jnp.min(jnp.where(w == m1, idx, E), axis=1, keepdims=True)
    w_rest = jnp.where(idx == i1, -jnp.inf, w)
    m2 = jnp.max(w_rest, axis=1, keepdims=True)
    i2 = jnp.min(jnp.where(w_rest == m2, idx, E), axis=1, keepdims=True)
    mask = (idx == i1) | (idx == i2)
    wsel = jnp.where(mask, w, 0.0)
    wn = wsel / (jnp.sum(wsel, axis=1, keepdims=True) + 1e-8)
    out_ref[...] = jnp.broadcast_to(wn.reshape(E, 1, 1), (E, H, W))[None]


@jax.jit
def kernel(x, W1, b1, W2, b2):
    xt = jnp.transpose(x, (0, 2, 3, 1))
    out = pl.pallas_call(
        _body,
        grid=(B,),
        in_specs=[
            pl.BlockSpec((1, H, W, C), lambda i: (i, 0, 0, 0)),
            pl.BlockSpec((C, RED), lambda i: (0, 0)),
            pl.BlockSpec((1, RED), lambda i: (0, 0)),
            pl.BlockSpec((RED, E), lambda i: (0, 0)),
            pl.BlockSpec((1, E), lambda i: (0, 0)),
        ],
        out_specs=pl.BlockSpec((1, E, H, W), lambda i: (i, 0, 0, 0)),
        out_shape=jax.ShapeDtypeStruct((B, E, H, W), jnp.float32),
    )(xt, W1, b1.reshape(1, RED), W2, b2.reshape(1, E))
    return out


# speedup vs baseline: 14.5770x; 1.9793x over previous
import jax
import jax.numpy as jnp
from jax import lax
from jax.experimental import pallas as pl

B, C, H, W = 64, 384, 32, 32
HW = H * W
E = 8
RED = 48


BB = 4


def _body(x_ref, w1_ref, b1_ref, w2_ref, b2_ref, out_ref):
    xs = x_ref[...]
    n = H
    while n > 1:
        n //= 2
        xs = xs[:, :n] + xs[:, n:2 * n]
    ys = xs[:, 0]
    n = W
    while n > 1:
        n //= 2
        ys = ys[:, :n] + ys[:, n:2 * n]
    pooled = ys[:, 0] * (1.0 / HW)
    h = jnp.dot(pooled, w1_ref[...], preferred_element_type=jnp.float32)
    h = h + b1_ref[...]
    h = h * jax.nn.sigmoid(h)
    logits = jnp.dot(h, w2_ref[...], preferred_element_type=jnp.float32)
    logits = logits + b2_ref[...]
    w = jax.nn.softmax(logits, axis=1)
    idx = lax.broadcasted_iota(jnp.int32, (BB, E), 1)
    m1 = jnp.max(w, axis=1, keepdims=True)
    i1 = jnp.min(jnp.where(w == m1, idx, E), axis=1, keepdims=True)
    w_rest = jnp.where(idx == i1, -jnp.inf, w)
    m2 = jnp.max(w_rest, axis=1, keepdims=True)
    i2 = jnp.min(jnp.where(w_rest == m2, idx, E), axis=1, keepdims=True)
    mask = (idx == i1) | (idx == i2)
    wsel = jnp.where(mask, w, 0.0)
    wn = wsel / (jnp.sum(wsel, axis=1, keepdims=True) + 1e-8)
    out_ref[...] = jnp.broadcast_to(
        wn.reshape(BB, E, 1, 1), (BB, E, H, W))


@jax.jit
def kernel(x, W1, b1, W2, b2):
    xt = jnp.transpose(x, (0, 2, 3, 1))
    out = pl.pallas_call(
        _body,
        grid=(B // BB,),
        in_specs=[
            pl.BlockSpec((BB, H, W, C), lambda i: (i, 0, 0, 0)),
            pl.BlockSpec((C, RED), lambda i: (0, 0)),
            pl.BlockSpec((1, RED), lambda i: (0, 0)),
            pl.BlockSpec((RED, E), lambda i: (0, 0)),
            pl.BlockSpec((1, E), lambda i: (0, 0)),
        ],
        out_specs=pl.BlockSpec((BB, E, H, W), lambda i: (i, 0, 0, 0)),
        out_shape=jax.ShapeDtypeStruct((B, E, H, W), jnp.float32),
    )(xt, W1, b1.reshape(1, RED), W2, b2.reshape(1, E))
    return out


# speedup vs baseline: 15.8655x; 1.0884x over previous
import jax
import jax.numpy as jnp
from jax import lax
from jax.experimental import pallas as pl

B, C, H, W = 64, 384, 32, 32
HW = H * W
E = 8
RED = 48


BB = 8


def _body(x_ref, w1_ref, b1_ref, w2_ref, b2_ref, out_ref):
    xs = x_ref[...]
    n = H
    while n > 1:
        n //= 2
        xs = xs[:, :n] + xs[:, n:2 * n]
    ys = xs[:, 0]
    n = W
    while n > 1:
        n //= 2
        ys = ys[:, :n] + ys[:, n:2 * n]
    pooled = ys[:, 0] * (1.0 / HW)
    h = jnp.dot(pooled, w1_ref[...], preferred_element_type=jnp.float32)
    h = h + b1_ref[...]
    h = h * jax.nn.sigmoid(h)
    logits = jnp.dot(h, w2_ref[...], preferred_element_type=jnp.float32)
    logits = logits + b2_ref[...]
    w = jax.nn.softmax(logits, axis=1)
    idx = lax.broadcasted_iota(jnp.int32, (BB, E), 1)
    m1 = jnp.max(w, axis=1, keepdims=True)
    i1 = jnp.min(jnp.where(w == m1, idx, E), axis=1, keepdims=True)
    w_rest = jnp.where(idx == i1, -jnp.inf, w)
    m2 = jnp.max(w_rest, axis=1, keepdims=True)
    i2 = jnp.min(jnp.where(w_rest == m2, idx, E), axis=1, keepdims=True)
    mask = (idx == i1) | (idx == i2)
    wsel = jnp.where(mask, w, 0.0)
    wn = wsel / (jnp.sum(wsel, axis=1, keepdims=True) + 1e-8)
    out_ref[...] = jnp.broadcast_to(
        wn.reshape(BB, E, 1, 1), (BB, E, H, W))


@jax.jit
def kernel(x, W1, b1, W2, b2):
    xt = jnp.transpose(x, (0, 2, 3, 1))
    out = pl.pallas_call(
        _body,
        grid=(B // BB,),
        in_specs=[
            pl.BlockSpec((BB, H, W, C), lambda i: (i, 0, 0, 0)),
            pl.BlockSpec((C, RED), lambda i: (0, 0)),
            pl.BlockSpec((1, RED), lambda i: (0, 0)),
            pl.BlockSpec((RED, E), lambda i: (0, 0)),
            pl.BlockSpec((1, E), lambda i: (0, 0)),
        ],
        out_specs=pl.BlockSpec((BB, E, H, W), lambda i: (i, 0, 0, 0)),
        out_shape=jax.ShapeDtypeStruct((B, E, H, W), jnp.float32),
    )(xt, W1, b1.reshape(1, RED), W2, b2.reshape(1, E))
    return out


# speedup vs baseline: 19.4787x; 1.2277x over previous
import jax
import jax.numpy as jnp
from jax import lax
from jax.experimental import pallas as pl
from jax.experimental.pallas import tpu as pltpu

B, C, H, W = 64, 384, 32, 32
HW = H * W
E = 8
RED = 48
BB = 8


def _body(x_ref, w1_ref, b1_ref, w2_ref, b2_ref, out_ref, wn_ref):
    i = pl.program_id(0)
    xs = x_ref[...]
    n = H
    while n > 1:
        n //= 2
        xs = xs[:, :n] + xs[:, n:2 * n]
    ys = xs[:, 0]
    n = W
    while n > 1:
        n //= 2
        ys = ys[:, :n] + ys[:, n:2 * n]
    pooled = ys[:, 0] * (1.0 / HW)
    h = jnp.dot(pooled, w1_ref[...], preferred_element_type=jnp.float32)
    h = h + b1_ref[...]
    h = h * jax.nn.sigmoid(h)
    logits = jnp.dot(h, w2_ref[...], preferred_element_type=jnp.float32)
    logits = logits + b2_ref[...]
    w = jax.nn.softmax(logits, axis=1)
    idx = lax.broadcasted_iota(jnp.int32, (BB, E), 1)
    m1 = jnp.max(w, axis=1, keepdims=True)
    i1 = jnp.min(jnp.where(w == m1, idx, E), axis=1, keepdims=True)
    w_rest = jnp.where(idx == i1, -jnp.inf, w)
    m2 = jnp.max(w_rest, axis=1, keepdims=True)
    i2 = jnp.min(jnp.where(w_rest == m2, idx, E), axis=1, keepdims=True)
    mask = (idx == i1) | (idx == i2)
    wsel = jnp.where(mask, w, 0.0)
    wn = wsel / (jnp.sum(wsel, axis=1, keepdims=True) + 1e-8)
    wn_ref[pl.ds(i * BB, BB), :] = wn

    @pl.when(i == B // BB - 1)
    def _():
        wnt = wn_ref[...].T
        out_ref[...] = jnp.broadcast_to(wnt[:, None, None, :], (E, H, W, B))


@jax.jit
def kernel(x, W1, b1, W2, b2):
    xt = jnp.transpose(x, (0, 2, 3, 1))
    pout = pl.pallas_call(
        _body,
        grid=(B // BB,),
        in_specs=[
            pl.BlockSpec((BB, H, W, C), lambda i: (i, 0, 0, 0)),
            pl.BlockSpec((C, RED), lambda i: (0, 0)),
            pl.BlockSpec((1, RED), lambda i: (0, 0)),
            pl.BlockSpec((RED, E), lambda i: (0, 0)),
            pl.BlockSpec((1, E), lambda i: (0, 0)),
        ],
        out_specs=pl.BlockSpec((E, H, W, B), lambda i: (0, 0, 0, 0)),
        out_shape=jax.ShapeDtypeStruct((E, H, W, B), jnp.float32),
        scratch_shapes=[pltpu.VMEM((B, E), jnp.float32)],
    )(xt, W1, b1.reshape(1, RED), W2, b2.reshape(1, E))
    return jnp.transpose(pout, (3, 0, 1, 2))
